# baseline (speedup 1.0000x reference)
import functools

import jax
import jax.numpy as jnp
from jax import lax
from jax.experimental import pallas as pl
from jax.experimental.pallas import tpu as pltpu
from jax.experimental.pallas import tpu_sc as plsc


@functools.lru_cache(maxsize=None)
def _make_copy(num_rows: int, dim: int, n_workers: int):
    rows_per_w = num_rows // n_workers
    mesh = plsc.VectorSubcoreMesh(core_axis_name="c", subcore_axis_name="s")

    @functools.partial(
        pl.kernel,
        out_type=jax.ShapeDtypeStruct((num_rows, dim), jnp.float32),
        mesh=mesh,
    )
    def copy_kernel(table_hbm, out_hbm):
        wid = lax.axis_index("s") * 2 + lax.axis_index("c")
        base = wid * rows_per_w
        pltpu.sync_copy(
            table_hbm.at[pl.ds(base, rows_per_w)],
            out_hbm.at[pl.ds(base, rows_per_w)],
        )

    return copy_kernel


def kernel(x, table):
    num_rows, dim = table.shape
    out = _make_copy(num_rows, dim, 32)(table)
    return out[None]


# speedup vs baseline: 24.8202x; 24.8202x over previous
import functools

import jax
import jax.numpy as jnp
from jax import lax
from jax.experimental import pallas as pl
from jax.experimental.pallas import tpu as pltpu
from jax.experimental.pallas import tpu_sc as plsc

_N_WORKERS = 32
_CHUNK_ROWS = 32
_N_BUF = 3


@functools.lru_cache(maxsize=None)
def _make_copy(num_rows: int, dim: int):
    rows_per_w = num_rows // _N_WORKERS
    n_chunks = rows_per_w // _CHUNK_ROWS
    mesh = plsc.VectorSubcoreMesh(core_axis_name="c", subcore_axis_name="s")

    @functools.partial(
        pl.kernel,
        out_type=jax.ShapeDtypeStruct((num_rows, dim), jnp.float32),
        mesh=mesh,
        scratch_types=(
            [pltpu.VMEM((_CHUNK_ROWS, dim), jnp.float32) for _ in range(_N_BUF)]
            + [pltpu.SemaphoreType.DMA for _ in range(2 * _N_BUF)]
        ),
    )
    def copy_kernel(table_hbm, out_hbm, *scratch):
        bufs = scratch[:_N_BUF]
        in_sems = scratch[_N_BUF:2 * _N_BUF]
        out_sems = scratch[2 * _N_BUF:]
        wid = lax.axis_index("s") * 2 + lax.axis_index("c")
        base = wid * rows_per_w

        def src(c):
            return table_hbm.at[pl.ds(base + c * _CHUNK_ROWS, _CHUNK_ROWS)]

        def dst(c):
            return out_hbm.at[pl.ds(base + c * _CHUNK_ROWS, _CHUNK_ROWS)]

        for b in range(min(_N_BUF, n_chunks)):
            pltpu.async_copy(src(b), bufs[b], in_sems[b])
        for c in range(n_chunks):
            b = c % _N_BUF
            pltpu.make_async_copy(src(c), bufs[b], in_sems[b]).wait()
            pltpu.async_copy(bufs[b], dst(c), out_sems[b])
            nxt = c + _N_BUF
            if nxt < n_chunks:
                pltpu.make_async_copy(bufs[b], dst(c), out_sems[b]).wait()
                pltpu.async_copy(src(nxt), bufs[b], in_sems[b])
        for c in range(max(0, n_chunks - _N_BUF), n_chunks):
            b = c % _N_BUF
            pltpu.make_async_copy(bufs[b], dst(c), out_sems[b]).wait()

    return copy_kernel


def kernel(x, table):
    num_rows, dim = table.shape
    out = _make_copy(num_rows, dim)(table)
    return out[None]
